# baseline (speedup 1.0000x reference)
import functools

import jax
import jax.numpy as jnp
from jax.experimental import pallas as pl
from jax.experimental.pallas import tpu as pltpu

B, N, D = 4, 2048, 128
UNITS = 128
RB = 512


def _body(x_ref, adj_ref, wm_ref, wu_ref, out_ref, msg_ref):
    rb = pl.program_id(1)

    @pl.when(rb == 0)
    def _compute_msg():
        msg_ref[...] = jnp.dot(
            x_ref[0], wm_ref[...], preferred_element_type=jnp.float32
        )

    a = adj_ref[0].astype(jnp.float32)
    agg = jax.lax.dot_general(
        a,
        msg_ref[...],
        (((0,), (0,)), ((), ())),
        preferred_element_type=jnp.float32,
    )
    deg = jnp.sum(a, axis=0)
    start = pl.multiple_of(rb * RB, RB)
    xr = x_ref[0, pl.ds(start, RB), :]
    upd = jnp.dot(xr, wu_ref[...], preferred_element_type=jnp.float32)
    mean = jnp.where(
        deg[:, None] > 0.0, agg / jnp.maximum(deg[:, None], 1.0), 0.0
    )
    out_ref[0] = upd + mean


@jax.jit
def kernel(x, adj, W_msg, W_upd):
    grid = (B, N // RB)
    return pl.pallas_call(
        _body,
        grid=grid,
        in_specs=[
            pl.BlockSpec((1, N, D), lambda b, r: (b, 0, 0)),
            pl.BlockSpec((1, N, RB), lambda b, r: (b, 0, r)),
            pl.BlockSpec((D, UNITS), lambda b, r: (0, 0)),
            pl.BlockSpec((D, UNITS), lambda b, r: (0, 0)),
        ],
        out_specs=pl.BlockSpec((1, RB, UNITS), lambda b, r: (b, r, 0)),
        out_shape=jax.ShapeDtypeStruct((B, N, UNITS), jnp.float32),
        scratch_shapes=[pltpu.VMEM((N, UNITS), jnp.float32)],
    )(x, adj, W_msg, W_upd)
